# baseline (speedup 1.0000x reference)
import jax
import jax.numpy as jnp
from jax import lax
from jax.experimental import pallas as pl
from jax.experimental.pallas import tpu as pltpu
from jax.experimental.pallas import tpu_sc as plsc

VOCAB_SIZE = 1000000
EMBED_DIM = 32
BATCH = 16384
HIST = 50
LANES = 128

_info = plsc.get_sparse_core_info()
NC = _info.num_cores
NS = _info.num_subcores
NW = NC * NS

B_PER_W = BATCH // NW
CHUNK = 128
N_CHUNKS = B_PER_W // CHUNK
DT = EMBED_DIM // 8


def _embed_body(idx_hbm, table_hbm, out_hbm, idx_v, wide_v, tiles_v, gsems, wsems):
    wid = lax.axis_index("s") * NC + lax.axis_index("c")
    base = wid * B_PER_W

    for c in range(N_CHUNKS):
        pltpu.sync_copy(
            idx_hbm.at[:, pl.ds(base + c * CHUNK, CHUNK)], idx_v.at[c]
        )

    iota16 = lax.iota(jnp.int32, 16)

    def issue_gather(h, c, q):
        pltpu.async_copy(
            table_hbm.at[idx_v.at[c, h]], wide_v.at[q], gsems[q]
        )

    def drain_gather(q):
        pltpu.make_async_copy(
            table_hbm.at[pl.ds(0, CHUNK)], wide_v.at[q], gsems[q]
        ).wait()

    def issue_writeout(h, p):
        for tr in range(DT):
            pltpu.async_copy(
                tiles_v.at[p, tr],
                out_hbm.at[h, tr, pl.ds(wid * N_CHUNKS, N_CHUNKS)],
                wsems[p],
            )

    def drain_writeout(p):
        pltpu.make_async_copy(
            table_hbm.at[pl.ds(0, CHUNK)], wide_v.at[0], wsems[p]
        ).wait()

    def transpose_chunk(q, p, c):
        qv = jnp.full((16,), q, jnp.int32)

        def kstep(k, carry):
            bvec = k * 16 + iota16
            for tr in range(DT):
                for ds in range(8):
                    vec = plsc.load_gather(
                        wide_v, [qv, bvec, jnp.full((16,), tr * 8 + ds, jnp.int32)]
                    )
                    tiles_v[p, tr, c, ds, pl.ds(k * 16, 16)] = vec
            return carry

        lax.fori_loop(0, CHUNK // 16, kstep, 0, unroll=False)

    def chunk_step(h, c, p, first_pair, last):
        q = c % 2
        drain_gather(q)
        if not last:
            nh, nc = (h, c + 1) if c + 1 < N_CHUNKS else (h + 1, 0)
            issue_gather(nh, nc, 1 - q)
        if c == 0 and not first_pair:
            drain_writeout(p)
        transpose_chunk(q, p, c)
        if c == N_CHUNKS - 1:
            issue_writeout(h, p)

    issue_gather(0, 0, 0)
    for h in (0, 1):
        for c in range(N_CHUNKS):
            chunk_step(h, c, p=h % 2, first_pair=True, last=False)

    def pair_step(t, carry):
        for hb in range(2):
            h = 2 * t + hb
            for c in range(N_CHUNKS):
                chunk_step(h, c, p=hb, first_pair=False, last=False)
        return carry

    lax.fori_loop(1, HIST // 2 - 1, pair_step, 0, unroll=False)

    for h in (HIST - 2, HIST - 1):
        for c in range(N_CHUNKS):
            chunk_step(h, c, p=h % 2, first_pair=False,
                       last=(h == HIST - 1 and c == N_CHUNKS - 1))
    drain_writeout(0)
    drain_writeout(1)


@jax.jit
def _embed(word_t, table_wide):
    mesh = plsc.VectorSubcoreMesh(core_axis_name="c", subcore_axis_name="s")
    k = pl.kernel(
        _embed_body,
        out_type=jax.ShapeDtypeStruct((HIST, DT, BATCH // CHUNK, 8, LANES),
                                      jnp.float32),
        mesh=mesh,
        scratch_types=[
            pltpu.VMEM((N_CHUNKS, HIST, CHUNK), jnp.int32),
            pltpu.VMEM((2, CHUNK, LANES), jnp.float32),
            pltpu.VMEM((2, DT, N_CHUNKS, 8, LANES), jnp.float32),
            [pltpu.SemaphoreType.DMA] * 2,
            [pltpu.SemaphoreType.DMA] * 2,
        ],
        compiler_params=pltpu.CompilerParams(
            use_tc_tiling_on_sc=False, needs_layout_passes=False
        ),
    )
    return k(word_t, table_wide)


def kernel(word, table):
    word_t = word.T.astype(jnp.int32)
    table_wide = jnp.pad(table, ((0, 0), (0, LANES - EMBED_DIM)))
    out5 = _embed(word_t, table_wide)
    return out5.transpose(2, 4, 0, 1, 3).reshape(BATCH, HIST, EMBED_DIM)


# speedup vs baseline: 1.1761x; 1.1761x over previous
import jax
import jax.numpy as jnp
from jax import lax
from jax.experimental import pallas as pl
from jax.experimental.pallas import tpu as pltpu
from jax.experimental.pallas import tpu_sc as plsc

VOCAB_SIZE = 1000000
EMBED_DIM = 32
BATCH = 16384
HIST = 50

_info = plsc.get_sparse_core_info()
NC = _info.num_cores
NS = _info.num_subcores
NW = NC * NS

B_PER_W = BATCH // NW
CHUNK = 128
N_CHUNKS = B_PER_W // CHUNK


def _embed_body(idx_hbm, table_hbm, out_hbm, idx_v, rows_v, gsems, wsems):
    wid = lax.axis_index("s") * NC + lax.axis_index("c")
    base = wid * B_PER_W

    for c in range(N_CHUNKS):
        pltpu.sync_copy(
            idx_hbm.at[:, pl.ds(base + c * CHUNK, CHUNK)], idx_v.at[c]
        )

    def issue_gathers(h, b):
        for c in range(N_CHUNKS):
            pltpu.async_copy(
                table_hbm.at[idx_v.at[c, h]],
                rows_v.at[b, pl.ds(c * CHUNK, CHUNK)],
                gsems[b],
            )

    def drain_gathers(b):
        pltpu.make_async_copy(
            table_hbm.at[pl.ds(0, B_PER_W), pl.ds(0, EMBED_DIM)],
            rows_v.at[b],
            gsems[b],
        ).wait()

    def issue_writeout(h, b):
        pltpu.async_copy(
            rows_v.at[b], out_hbm.at[h, pl.ds(base, B_PER_W)], wsems[b]
        )

    def drain_writeout(b):
        pltpu.make_async_copy(
            table_hbm.at[pl.ds(0, B_PER_W), pl.ds(0, EMBED_DIM)],
            rows_v.at[b],
            wsems[b],
        ).wait()

    def body(h, bh, b, ahead, first):
        if ahead:
            if not first:
                drain_writeout(b)
            issue_gathers(h + 2, b)
        drain_gathers(bh)
        issue_writeout(h, bh)

    issue_gathers(0, 0)
    issue_gathers(1, 1)
    body(0, 0, 2, ahead=True, first=True)
    body(1, 1, 3, ahead=True, first=True)

    def quad_step(t, carry):
        for b in range(4):
            h = 4 * t + 2 + b
            body(h, (2 + b) % 4, b, ahead=True, first=False)
        return carry

    lax.fori_loop(0, (HIST - 6) // 4, quad_step, 0, unroll=False)

    for b in range(4):
        h = HIST - 4 + b
        body(h, (2 + b) % 4, b, ahead=(h + 2 < HIST), first=False)
    for h in range(HIST - 4, HIST):
        drain_writeout((2 + h - (HIST - 4)) % 4)


@jax.jit
def _embed(word_t, table):
    mesh = plsc.VectorSubcoreMesh(core_axis_name="c", subcore_axis_name="s")
    k = pl.kernel(
        _embed_body,
        out_type=jax.ShapeDtypeStruct((HIST, BATCH, EMBED_DIM), jnp.float32),
        mesh=mesh,
        scratch_types=[
            pltpu.VMEM((N_CHUNKS, HIST, CHUNK), jnp.int32),
            pltpu.VMEM((4, B_PER_W, EMBED_DIM), jnp.float32),
            [pltpu.SemaphoreType.DMA] * 4,
            [pltpu.SemaphoreType.DMA] * 4,
        ],
        compiler_params=pltpu.CompilerParams(use_tc_tiling_on_sc=False),
    )
    return k(word_t, table)


def kernel(word, table):
    word_t = word.T.astype(jnp.int32)
    out = _embed(word_t, table)
    return jnp.transpose(out, (1, 0, 2))
